# Initial kernel scaffold; baseline (speedup 1.0000x reference)
import jax
import jax.numpy as jnp
from jax.experimental import pallas as pl


def kernel(x, edge_index, params):
    raise NotImplementedError("write your pallas kernel here")


# speedup vs baseline: 26.6179x; 26.6179x over previous
import functools

import jax
import jax.numpy as jnp
import numpy as np
from jax import lax
from jax.experimental import pallas as pl
from jax.experimental.pallas import tpu as pltpu
from jax.experimental.pallas import tpu_sc as plsc

N = 10000
E = 320000
D_IN = 128
H = 64

NC = 2
NS = 16
NW = NC * NS
EW = E // NW
B = 80
S = EW // B
BA = 125
SA = EW // BA
NBUF = 8
RPS = 624
TAIL = N - NS * RPS

BN_SCALE = float(1.0 / np.sqrt(np.float32(1.0 + 1e-5)))

_MESH = plsc.VectorSubcoreMesh(
    core_axis_name="c", subcore_axis_name="s", num_cores=NC, num_subcores=NS)
_SC_PARAMS = pltpu.CompilerParams(use_tc_tiling_on_sc=False)


@functools.partial(
    pl.kernel,
    out_type=jax.ShapeDtypeStruct((NC, N), jnp.float32),
    mesh=_MESH,
    compiler_params=_SC_PARAMS,
    scratch_types=[
        pltpu.VMEM((S, B), jnp.int32),
        pltpu.VMEM((B,), jnp.float32),
        pltpu.VMEM_SHARED((N,), jnp.float32),
        [pltpu.SemaphoreType.DMA] * 5,
    ],
)
def _sc_degree(src_hbm, zeros_hbm, out_hbm, idx_v, ones_v, acc, sems):
    c = lax.axis_index("c")
    s = lax.axis_index("s")
    w = c * NS + s
    for i in range(B // 16):
        ones_v[pl.ds(i * 16, 16)] = jnp.ones((16,), jnp.float32)
    @pl.when(s == 0)
    def _():
        pltpu.sync_copy(zeros_hbm, acc)
    pltpu.sync_copy(src_hbm.at[w], idx_v)
    plsc.subcore_barrier()

    def body(i, carry):
        base = i * 5
        sds = [pltpu.async_copy(ones_v, acc.at[idx_v.at[base + b]],
                                sems[b], add=True) for b in range(5)]
        for sd in sds:
            sd.wait()
        return carry

    lax.fori_loop(0, S // 5, body, 0)
    plsc.subcore_barrier()
    @pl.when(s == 0)
    def _():
        pltpu.sync_copy(acc, out_hbm.at[c])


@functools.partial(
    pl.kernel,
    out_type=jax.ShapeDtypeStruct((NC, N, H), jnp.float32),
    mesh=_MESH,
    compiler_params=_SC_PARAMS,
    scratch_types=[
        pltpu.VMEM((SA, BA), jnp.int32),
        pltpu.VMEM((SA, BA), jnp.int32),
        [pltpu.VMEM((BA, H), jnp.float32)] * NBUF,
        [pltpu.SemaphoreType.DMA] * NBUF,
        [pltpu.SemaphoreType.DMA] * NBUF,
        pltpu.VMEM_SHARED((N, H), jnp.float32),
    ],
)
def _sc_edge_agg(y_hbm, src_hbm, dst_hbm, zeros_hbm, out_hbm,
                 src_v, dst_v, rows, gsems, ssems, acc):
    c = lax.axis_index("c")
    s = lax.axis_index("s")
    w = c * NS + s
    pltpu.sync_copy(zeros_hbm.at[pl.ds(s * RPS, RPS)],
                    acc.at[pl.ds(s * RPS, RPS)])
    @pl.when(s == 0)
    def _():
        pltpu.sync_copy(zeros_hbm.at[pl.ds(NS * RPS, TAIL)],
                        acc.at[pl.ds(NS * RPS, TAIL)])
    pltpu.sync_copy(src_hbm.at[w], src_v)
    pltpu.sync_copy(dst_hbm.at[w], dst_v)
    plsc.subcore_barrier()

    def body(i, carry):
        base = i * NBUF
        gds = [pltpu.async_copy(y_hbm.at[src_v.at[base + b]], rows[b],
                                gsems[b]) for b in range(NBUF)]
        sds = []
        for b in range(NBUF):
            gds[b].wait()
            sds.append(pltpu.async_copy(rows[b], acc.at[dst_v.at[base + b]],
                                        ssems[b], add=True))
        for b in range(NBUF):
            sds[b].wait()
        return carry

    lax.fori_loop(0, SA // NBUF, body, 0)
    plsc.subcore_barrier()
    pltpu.sync_copy(acc.at[pl.ds(s * RPS, RPS)],
                    out_hbm.at[c].at[pl.ds(s * RPS, RPS)])
    @pl.when(s == 0)
    def _():
        pltpu.sync_copy(acc.at[pl.ds(NS * RPS, TAIL)],
                        out_hbm.at[c].at[pl.ds(NS * RPS, TAIL)])


R = 2000
G = N // R
_TC_PARAMS = pltpu.CompilerParams(dimension_semantics=("parallel",))


def _bcast(shape):
    return pl.BlockSpec(shape, lambda i: (0,) * len(shape))


def _rows(*minor):
    nm = len(minor)
    return pl.BlockSpec((R,) + minor, lambda i: (i,) + (0,) * nm)


def _rows3(*minor):
    nm = len(minor)
    return pl.BlockSpec((2, R) + minor, lambda i: (0, i) + (0,) * nm)


def _tc0_body(deg_ref, x_ref, w0b_ref, dinv_ref, y0_ref):
    deg = deg_ref[0] + deg_ref[1]
    dinv = jnp.where(deg > 0.0,
                     lax.rsqrt(jnp.maximum(deg, 1e-12)),
                     0.0)
    dinv_ref[...] = dinv
    y0_ref[...] = jnp.dot(x_ref[...], w0b_ref[...],
                          preferred_element_type=jnp.float32) * dinv


_tc0 = pl.pallas_call(
    _tc0_body,
    grid=(G,),
    in_specs=[_rows3(1), _rows(D_IN), _bcast((D_IN, H))],
    out_specs=(_rows(1), _rows(H)),
    out_shape=(
        jax.ShapeDtypeStruct((N, 1), jnp.float32),
        jax.ShapeDtypeStruct((N, H), jnp.float32),
    ),
    compiler_params=_TC_PARAMS,
)


def _tc_mid_body(sp_ref, hin_ref, dinv_ref, g_ref, bb_ref, bias_ref,
                 wa_ref, wbn_ref, h_ref, y2_ref, *, residual):
    dinv = dinv_ref[...]
    hin = hin_ref[...]
    t = jnp.dot(hin, wa_ref[...], preferred_element_type=jnp.float32)
    msg = -dinv * (sp_ref[0] + sp_ref[1])
    pre = t + msg + bias_ref[...]
    h = jnp.maximum(pre * (g_ref[...] * BN_SCALE) + bb_ref[...], 0.0)
    if residual:
        h = h + hin
    h_ref[...] = h
    y2_ref[...] = jnp.dot(h, wbn_ref[...],
                          preferred_element_type=jnp.float32) * dinv


def _make_tc_mid(din, residual):
    return pl.pallas_call(
        functools.partial(_tc_mid_body, residual=residual),
        grid=(G,),
        in_specs=[_rows3(H), _rows(din), _rows(1), _bcast((1, H)),
                  _bcast((1, H)), _bcast((1, H)),
                  _bcast((din, H)), _bcast((H, H))],
        out_specs=(_rows(H), _rows(H)),
        out_shape=(
            jax.ShapeDtypeStruct((N, H), jnp.float32),
            jax.ShapeDtypeStruct((N, H), jnp.float32),
        ),
        compiler_params=_TC_PARAMS,
    )


_tc_mid0 = _make_tc_mid(D_IN, False)
_tc_mid1 = _make_tc_mid(H, True)


def _ln(h, g, b):
    m = jnp.mean(h, axis=-1, keepdims=True)
    v = jnp.mean((h - m) * (h - m), axis=-1, keepdims=True)
    return (h - m) * lax.rsqrt(v + 1e-5) * g + b


def _tc_final_body(sp_ref, hin_ref, dinv_ref, g_ref, bb_ref, bias_ref,
                   wa_ref, pmw1_ref, pmb1_ref, pmlng_ref, pmlnb_ref, pmw2_ref,
                   pmb2_ref, rmw1_ref, rmb1_ref, rmlng_ref, rmlnb_ref,
                   rmw2_ref, rmb2_ref, out_ref):
    dinv = dinv_ref[...]
    hin = hin_ref[...]
    t = jnp.dot(hin, wa_ref[...], preferred_element_type=jnp.float32)
    msg = -dinv * (sp_ref[0] + sp_ref[1])
    pre = t + msg + bias_ref[...]
    h2 = jnp.maximum(pre * (g_ref[...] * BN_SCALE) + bb_ref[...], 0.0)
    h2 = h2 + hin
    p1 = jnp.dot(h2, pmw1_ref[...],
                 preferred_element_type=jnp.float32) + pmb1_ref[...]
    p1 = jnp.maximum(_ln(p1, pmlng_ref[...], pmlnb_ref[...]), 0.0)
    pos = jnp.dot(p1, pmw2_ref[...],
                  preferred_element_type=jnp.float32) + pmb2_ref[...]
    r1 = jnp.dot(h2, rmw1_ref[...],
                 preferred_element_type=jnp.float32) + rmb1_ref[...]
    r1 = jnp.maximum(_ln(r1, rmlng_ref[...], rmlnb_ref[...]), 0.0)
    rad = jax.nn.sigmoid(jnp.dot(r1, rmw2_ref[...],
                                 preferred_element_type=jnp.float32)
                         + rmb2_ref[...])
    nrm = jnp.sqrt(jnp.sum(pos * pos, axis=-1, keepdims=True))
    out_ref[...] = pos / (nrm + 1e-8) * rad


_tc_final = pl.pallas_call(
    _tc_final_body,
    grid=(G,),
    in_specs=[_rows3(H), _rows(H), _rows(1), _bcast((1, H)), _bcast((1, H)),
              _bcast((1, H)), _bcast((H, H)),
              _bcast((H, H)), _bcast((1, H)), _bcast((1, H)), _bcast((1, H)),
              _bcast((H, 2)), _bcast((1, 2)),
              _bcast((H, H // 2)), _bcast((1, H // 2)), _bcast((1, H // 2)),
              _bcast((1, H // 2)), _bcast((H // 2, 1)), _bcast((1, 1))],
    out_specs=pl.BlockSpec((R, 2), lambda i: (i, 0)),
    out_shape=jax.ShapeDtypeStruct((N, 2), jnp.float32),
    compiler_params=_TC_PARAMS,
)


def kernel(x, edge_index, params):
    p = params
    src = edge_index[0].reshape(NW, SA, BA)
    dst = edge_index[1].reshape(NW, SA, BA)
    src_d = edge_index[0].reshape(NW, S, B)
    z1 = jnp.zeros((N,), jnp.float32)
    z64 = jnp.zeros((N, H), jnp.float32)

    degp = _sc_degree(src_d, z1)

    row = lambda v: v.reshape(1, -1)
    dinv, y0 = _tc0(degp.reshape(NC, N, 1), x, p["W0b"])
    s0 = _sc_edge_agg(y0, src, dst, z64)
    h, y1 = _tc_mid0(s0, x, dinv, row(p["bn0_g"]), row(p["bn0_b"]),
                     row(p["b0"]), p["W0a"], p["W1b"])
    s1 = _sc_edge_agg(y1, src, dst, z64)
    h1, y2 = _tc_mid1(s1, h, dinv, row(p["bn1_g"]), row(p["bn1_b"]),
                      row(p["b1"]), p["W1a"], p["W2b"])
    s2 = _sc_edge_agg(y2, src, dst, z64)
    out = _tc_final(s2, h1, dinv, row(p["bn2_g"]), row(p["bn2_b"]),
                    row(p["b2"]), p["W2a"],
                    p["pm_w1"], row(p["pm_b1"]), row(p["pm_lng"]),
                    row(p["pm_lnb"]), p["pm_w2"], row(p["pm_b2"]),
                    p["rm_w1"], row(p["rm_b1"]), row(p["rm_lng"]),
                    row(p["rm_lnb"]), p["rm_w2"], row(p["rm_b2"]))
    return out
